# baseline (speedup 1.0000x reference)
import functools

import jax
import jax.numpy as jnp
from jax import lax
from jax.experimental import pallas as pl
from jax.experimental.pallas import tpu as pltpu
from jax.experimental.pallas import tpu_sc as plsc

N = 10000
E = 320000
D = 128
BN_EPS = 1e-5

NC = 2
NS = 16
NW = NC * NS

CHUNK = 128
NSET = 8
TPT = 80
E_PAD = NW * TPT * CHUNK
ACC_ROWS = N + 8

RPT = 624
RPT_LAST = N - 15 * RPT


def _seg_sum_body(h_hbm, srcb_hbm, dstb_hbm, zeros_hbm, out_hbm,
                  si0, si1, si2, si3, si4, si5, si6, si7,
                  di0, di1, di2, di3, di4, di5, di6, di7,
                  rows, acc, sem_i, sem_g):
    cid = lax.axis_index("c")
    sid = lax.axis_index("s")
    wid = cid * NS + sid
    sidx = [si0, si1, si2, si3, si4, si5, si6, si7]
    didx = [di0, di1, di2, di3, di4, di5, di6, di7]

    base = sid * RPT

    @pl.when(sid < NS - 1)
    def _():
        pltpu.sync_copy(zeros_hbm.at[pl.ds(base, RPT)],
                        acc.at[pl.ds(base, RPT)])

    @pl.when(sid == NS - 1)
    def _():
        pltpu.sync_copy(zeros_hbm.at[pl.ds(base, RPT_LAST)],
                        acc.at[pl.ds(base, RPT_LAST)])

    plsc.subcore_barrier()

    def body(g, carry):
        descrs = []
        for b in range(NSET):
            j = g * NSET + b
            descrs.append(pltpu.async_copy(srcb_hbm.at[wid, j], sidx[b],
                                           sem_i))
            descrs.append(pltpu.async_copy(dstb_hbm.at[wid, j], didx[b],
                                           sem_i))
        for b in range(NSET):
            descrs[2 * b].wait()
            descrs[2 * b + 1].wait()
            pltpu.async_copy(h_hbm.at[sidx[b]], rows, sem_g).wait()
            pltpu.sync_copy(rows, acc.at[didx[b]], add=True)
        return carry

    lax.fori_loop(0, TPT // NSET, body, 0)
    plsc.subcore_barrier()

    @pl.when(sid < NS - 1)
    def _():
        pltpu.sync_copy(acc.at[pl.ds(base, RPT)],
                        out_hbm.at[pl.ds(cid * N + base, RPT)])

    @pl.when(sid == NS - 1)
    def _():
        pltpu.sync_copy(acc.at[pl.ds(base, RPT_LAST)],
                        out_hbm.at[pl.ds(cid * N + base, RPT_LAST)])


_seg_sum = pl.kernel(
    _seg_sum_body,
    out_type=jax.ShapeDtypeStruct((NC * N, D), jnp.float32),
    mesh=plsc.VectorSubcoreMesh(core_axis_name="c", subcore_axis_name="s"),
    scratch_types=(
        [pltpu.VMEM((CHUNK,), jnp.int32)] * 16
        + [
            pltpu.VMEM((CHUNK, D), jnp.float32),
            pltpu.VMEM_SHARED((ACC_ROWS, D), jnp.float32),
            pltpu.SemaphoreType.DMA,
            pltpu.SemaphoreType.DMA,
        ]
    ),
)


def _bn(a, g, b):
    m = jnp.mean(a, axis=0)
    v = jnp.mean((a - m) * (a - m), axis=0)
    return (a - m) * lax.rsqrt(v + BN_EPS) * g + b


def _dense_body(h_ref, part_ref, w1_ref, b1_ref, gi_ref, bi_ref,
                w2_ref, b2_ref, go_ref, bo_ref, o_ref, *, relu_out):
    s = h_ref[...] + part_ref[:N] + part_ref[N:]
    a = jnp.dot(s, w1_ref[...], preferred_element_type=jnp.float32)
    a = a + b1_ref[...]
    a = jnp.maximum(_bn(a, gi_ref[...], bi_ref[...]), 0.0)
    o = jnp.dot(a, w2_ref[...], preferred_element_type=jnp.float32)
    o = o + b2_ref[...]
    o = _bn(o, go_ref[...], bo_ref[...])
    if relu_out:
        o = jnp.maximum(o, 0.0)
    o_ref[...] = o


def _dense(h, part, w1, b1, gi, bi, w2, b2, go, bo, relu_out):
    return pl.pallas_call(
        functools.partial(_dense_body, relu_out=relu_out),
        out_shape=jax.ShapeDtypeStruct((N, D), jnp.float32),
    )(h, part, w1, b1, gi, bi, w2, b2, go, bo)


def kernel(x, edge_index, batch,
           w1_0, b1_0, gi_0, bi_0, w2_0, b2_0, go_0, bo_0,
           w1_1, b1_1, gi_1, bi_1, w2_1, b2_1, go_1, bo_1,
           w1_2, b1_2, gi_2, bi_2, w2_2, b2_2, go_2, bo_2):
    src = edge_index[0]
    dst = edge_index[1]
    pad = E_PAD - E
    srcb = jnp.concatenate([src, jnp.zeros((pad,), jnp.int32)])
    srcb = srcb.reshape(NW, TPT, CHUNK)
    dstb = jnp.concatenate([dst, jnp.full((pad,), N, jnp.int32)])
    dstb = dstb.reshape(NW, TPT, CHUNK)
    zeros = jnp.zeros((N, D), jnp.float32)

    params = [
        (w1_0, b1_0, gi_0, bi_0, w2_0, b2_0, go_0, bo_0),
        (w1_1, b1_1, gi_1, bi_1, w2_1, b2_1, go_1, bo_1),
        (w1_2, b1_2, gi_2, bi_2, w2_2, b2_2, go_2, bo_2),
    ]

    h = x
    for l in range(3):
        part = _seg_sum(h, srcb, dstb, zeros)
        h = _dense(h, part, *params[l], relu_out=(l < 2))
    return h


# speedup vs baseline: 1.1691x; 1.1691x over previous
import functools

import jax
import jax.numpy as jnp
from jax import lax
from jax.experimental import pallas as pl
from jax.experimental.pallas import tpu as pltpu
from jax.experimental.pallas import tpu_sc as plsc

N = 10000
E = 320000
D = 128
BN_EPS = 1e-5

NC = 2
NS = 16
NW = NC * NS

CHUNK = 128
NSET = 8
TPT = 80
E_PAD = NW * TPT * CHUNK
EPT = E // NW
PPT = TPT * CHUNK - EPT
ACC_ROWS = N + CHUNK

RPT = 624
RPT_LAST = N - 15 * RPT


def _seg_sum_body(h_hbm, srcb_hbm, dstb_hbm, zeros_hbm, out_hbm,
                  si0, si1, si2, si3, si4, si5, si6, si7,
                  di0, di1, di2, di3, di4, di5, di6, di7,
                  rows, acc, sem_i, sem_g):
    cid = lax.axis_index("c")
    sid = lax.axis_index("s")
    wid = cid * NS + sid
    sidx = [si0, si1, si2, si3, si4, si5, si6, si7]
    didx = [di0, di1, di2, di3, di4, di5, di6, di7]

    base = sid * RPT

    @pl.when(sid < NS - 1)
    def _():
        pltpu.sync_copy(zeros_hbm.at[pl.ds(base, RPT)],
                        acc.at[pl.ds(base, RPT)])

    @pl.when(sid == NS - 1)
    def _():
        pltpu.sync_copy(zeros_hbm.at[pl.ds(base, RPT_LAST)],
                        acc.at[pl.ds(base, RPT_LAST)])

    plsc.subcore_barrier()

    def body(g, carry):
        descrs = []
        for b in range(NSET):
            j = g * NSET + b
            descrs.append(pltpu.async_copy(srcb_hbm.at[wid, j], sidx[b],
                                           sem_i))
            descrs.append(pltpu.async_copy(dstb_hbm.at[wid, j], didx[b],
                                           sem_i))
        for b in range(NSET):
            descrs[2 * b].wait()
            descrs[2 * b + 1].wait()
            pltpu.async_copy(h_hbm.at[sidx[b]], rows, sem_g).wait()
            pltpu.sync_copy(rows, acc.at[didx[b]], add=True)
        return carry

    lax.fori_loop(0, TPT // NSET, body, 0)
    plsc.subcore_barrier()

    @pl.when(sid < NS - 1)
    def _():
        pltpu.sync_copy(acc.at[pl.ds(base, RPT)],
                        out_hbm.at[pl.ds(cid * N + base, RPT)])

    @pl.when(sid == NS - 1)
    def _():
        pltpu.sync_copy(acc.at[pl.ds(base, RPT_LAST)],
                        out_hbm.at[pl.ds(cid * N + base, RPT_LAST)])


_seg_sum = pl.kernel(
    _seg_sum_body,
    out_type=jax.ShapeDtypeStruct((NC * N, D), jnp.float32),
    mesh=plsc.VectorSubcoreMesh(core_axis_name="c", subcore_axis_name="s"),
    scratch_types=(
        [pltpu.VMEM((CHUNK,), jnp.int32)] * 16
        + [
            pltpu.VMEM((CHUNK, D), jnp.float32),
            pltpu.VMEM_SHARED((ACC_ROWS, D), jnp.float32),
            pltpu.SemaphoreType.DMA,
            pltpu.SemaphoreType.DMA,
        ]
    ),
)


def _bn(a, g, b):
    m = jnp.mean(a, axis=0)
    v = jnp.mean((a - m) * (a - m), axis=0)
    return (a - m) * lax.rsqrt(v + BN_EPS) * g + b


def _dense_body(h_ref, part_ref, w1_ref, b1_ref, gi_ref, bi_ref,
                w2_ref, b2_ref, go_ref, bo_ref, o_ref, *, relu_out):
    s = h_ref[...] + part_ref[:N] + part_ref[N:]
    a = jnp.dot(s, w1_ref[...], preferred_element_type=jnp.float32)
    a = a + b1_ref[...]
    a = jnp.maximum(_bn(a, gi_ref[...], bi_ref[...]), 0.0)
    o = jnp.dot(a, w2_ref[...], preferred_element_type=jnp.float32)
    o = o + b2_ref[...]
    o = _bn(o, go_ref[...], bo_ref[...])
    if relu_out:
        o = jnp.maximum(o, 0.0)
    o_ref[...] = o


def _dense(h, part, w1, b1, gi, bi, w2, b2, go, bo, relu_out):
    return pl.pallas_call(
        functools.partial(_dense_body, relu_out=relu_out),
        out_shape=jax.ShapeDtypeStruct((N, D), jnp.float32),
    )(h, part, w1, b1, gi, bi, w2, b2, go, bo)


def kernel(x, edge_index, batch,
           w1_0, b1_0, gi_0, bi_0, w2_0, b2_0, go_0, bo_0,
           w1_1, b1_1, gi_1, bi_1, w2_1, b2_1, go_1, bo_1,
           w1_2, b1_2, gi_2, bi_2, w2_2, b2_2, go_2, bo_2):
    src = edge_index[0]
    dst = edge_index[1]
    spad = jnp.zeros((NW, PPT), jnp.int32)
    dpad = jnp.broadcast_to(N + (jnp.arange(PPT, dtype=jnp.int32) % CHUNK),
                            (NW, PPT))
    srcb = jnp.concatenate([src.reshape(NW, EPT), spad], axis=1)
    srcb = srcb.reshape(NW, TPT, CHUNK)
    dstb = jnp.concatenate([dst.reshape(NW, EPT), dpad], axis=1)
    dstb = dstb.reshape(NW, TPT, CHUNK)
    zeros = jnp.zeros((N, D), jnp.float32)

    params = [
        (w1_0, b1_0, gi_0, bi_0, w2_0, b2_0, go_0, bo_0),
        (w1_1, b1_1, gi_1, bi_1, w2_1, b2_1, go_1, bo_1),
        (w1_2, b1_2, gi_2, bi_2, w2_2, b2_2, go_2, bo_2),
    ]

    h = x
    for l in range(3):
        part = _seg_sum(h, srcb, dstb, zeros)
        h = _dense(h, part, *params[l], relu_out=(l < 2))
    return h


# speedup vs baseline: 2.8270x; 2.4180x over previous
import functools

import jax
import jax.numpy as jnp
from jax import lax
from jax.experimental import pallas as pl
from jax.experimental.pallas import tpu as pltpu
from jax.experimental.pallas import tpu_sc as plsc

N = 10000
E = 320000
D = 128
BN_EPS = 1e-5

NC = 2
NS = 16
NW = NC * NS

CHUNK = 128
NCHUNK = E // CHUNK
TPT = (NCHUNK + NW - 1) // NW
RPT = 624
RPT_LAST = N - 15 * RPT


def _seg_sum_body(h_hbm, src_hbm, dst_hbm, zeros_hbm, out_hbm,
                  sidx0, didx0, rows0, sidx1, didx1, rows1, acc,
                  sem0, sem1):
    cid = lax.axis_index("c")
    sid = lax.axis_index("s")
    wid = cid * NS + sid

    base = sid * RPT

    @pl.when(sid < NS - 1)
    def _():
        pltpu.sync_copy(zeros_hbm.at[pl.ds(base, RPT)],
                        acc.at[pl.ds(base, RPT)])

    @pl.when(sid == NS - 1)
    def _():
        pltpu.sync_copy(zeros_hbm.at[pl.ds(base, RPT_LAST)],
                        acc.at[pl.ds(base, RPT_LAST)])

    plsc.subcore_barrier()

    def step(g, carry):
        c0 = (2 * g) * NW + wid
        c1 = (2 * g + 1) * NW + wid

        @pl.when(c0 < NCHUNK)
        def _():
            off = pl.multiple_of(c0 * CHUNK, CHUNK)
            pltpu.sync_copy(src_hbm.at[pl.ds(off, CHUNK)], sidx0)
            pltpu.sync_copy(dst_hbm.at[pl.ds(off, CHUNK)], didx0)
            pltpu.async_copy(h_hbm.at[sidx0], rows0, sem0)

        @pl.when(c1 < NCHUNK)
        def _():
            off = pl.multiple_of(c1 * CHUNK, CHUNK)
            pltpu.sync_copy(src_hbm.at[pl.ds(off, CHUNK)], sidx1)
            pltpu.sync_copy(dst_hbm.at[pl.ds(off, CHUNK)], didx1)
            pltpu.async_copy(h_hbm.at[sidx1], rows1, sem1)

        @pl.when(c0 < NCHUNK)
        def _():
            pltpu.make_async_copy(h_hbm.at[sidx0], rows0, sem0).wait()
            pltpu.sync_copy(rows0, acc.at[didx0], add=True)

        @pl.when(c1 < NCHUNK)
        def _():
            pltpu.make_async_copy(h_hbm.at[sidx1], rows1, sem1).wait()
            pltpu.sync_copy(rows1, acc.at[didx1], add=True)

        return carry

    lax.fori_loop(0, (TPT + 1) // 2, step, 0)
    plsc.subcore_barrier()

    @pl.when(sid < NS - 1)
    def _():
        pltpu.sync_copy(acc.at[pl.ds(base, RPT)],
                        out_hbm.at[pl.ds(cid * N + base, RPT)])

    @pl.when(sid == NS - 1)
    def _():
        pltpu.sync_copy(acc.at[pl.ds(base, RPT_LAST)],
                        out_hbm.at[pl.ds(cid * N + base, RPT_LAST)])


_seg_sum = pl.kernel(
    _seg_sum_body,
    out_type=jax.ShapeDtypeStruct((NC * N, D), jnp.float32),
    mesh=plsc.VectorSubcoreMesh(core_axis_name="c", subcore_axis_name="s"),
    scratch_types=[
        pltpu.VMEM((CHUNK,), jnp.int32),
        pltpu.VMEM((CHUNK,), jnp.int32),
        pltpu.VMEM((CHUNK, D), jnp.float32),
        pltpu.VMEM((CHUNK,), jnp.int32),
        pltpu.VMEM((CHUNK,), jnp.int32),
        pltpu.VMEM((CHUNK, D), jnp.float32),
        pltpu.VMEM_SHARED((N, D), jnp.float32),
        pltpu.SemaphoreType.DMA,
        pltpu.SemaphoreType.DMA,
    ],
)


def _bn(a, g, b):
    m = jnp.mean(a, axis=0)
    v = jnp.mean((a - m) * (a - m), axis=0)
    return (a - m) * lax.rsqrt(v + BN_EPS) * g + b


def _dense_body(h_ref, part_ref, w1_ref, b1_ref, gi_ref, bi_ref,
                w2_ref, b2_ref, go_ref, bo_ref, o_ref, *, relu_out):
    s = h_ref[...] + part_ref[:N] + part_ref[N:]
    a = jnp.dot(s, w1_ref[...], preferred_element_type=jnp.float32)
    a = a + b1_ref[...]
    a = jnp.maximum(_bn(a, gi_ref[...], bi_ref[...]), 0.0)
    o = jnp.dot(a, w2_ref[...], preferred_element_type=jnp.float32)
    o = o + b2_ref[...]
    o = _bn(o, go_ref[...], bo_ref[...])
    if relu_out:
        o = jnp.maximum(o, 0.0)
    o_ref[...] = o


def _dense(h, part, w1, b1, gi, bi, w2, b2, go, bo, relu_out):
    return pl.pallas_call(
        functools.partial(_dense_body, relu_out=relu_out),
        out_shape=jax.ShapeDtypeStruct((N, D), jnp.float32),
    )(h, part, w1, b1, gi, bi, w2, b2, go, bo)


def kernel(x, edge_index, batch,
           w1_0, b1_0, gi_0, bi_0, w2_0, b2_0, go_0, bo_0,
           w1_1, b1_1, gi_1, bi_1, w2_1, b2_1, go_1, bo_1,
           w1_2, b1_2, gi_2, bi_2, w2_2, b2_2, go_2, bo_2):
    src = edge_index[0]
    dst = edge_index[1]
    zeros = jnp.zeros((N, D), jnp.float32)

    params = [
        (w1_0, b1_0, gi_0, bi_0, w2_0, b2_0, go_0, bo_0),
        (w1_1, b1_1, gi_1, bi_1, w2_1, b2_1, go_1, bo_1),
        (w1_2, b1_2, gi_2, bi_2, w2_2, b2_2, go_2, bo_2),
    ]

    h = x
    for l in range(3):
        part = _seg_sum(h, src, dst, zeros)
        h = _dense(h, part, *params[l], relu_out=(l < 2))
    return h


# speedup vs baseline: 3.1676x; 1.1205x over previous
import functools

import jax
import jax.numpy as jnp
from jax import lax
from jax.experimental import pallas as pl
from jax.experimental.pallas import tpu as pltpu
from jax.experimental.pallas import tpu_sc as plsc

N = 10000
E = 320000
D = 128
BN_EPS = 1e-5

NC = 2
NS = 16
NW = NC * NS

CHUNK = 128
NCHUNK = E // CHUNK
TPT = (NCHUNK + NW - 1) // NW
RPT = 624
RPT_LAST = N - 15 * RPT


def _seg_sum_body(h_hbm, src_hbm, dst_hbm, zeros_hbm, out_hbm,
                  sidx0, didx0, rows0, sidx1, didx1, rows1,
                  sidx2, didx2, rows2, acc, sem0, sem1, sem2):
    cid = lax.axis_index("c")
    sid = lax.axis_index("s")
    wid = cid * NS + sid
    sets = [(sidx0, didx0, rows0, sem0),
            (sidx1, didx1, rows1, sem1),
            (sidx2, didx2, rows2, sem2)]

    base = sid * RPT

    @pl.when(sid < NS - 1)
    def _():
        pltpu.sync_copy(zeros_hbm.at[pl.ds(base, RPT)],
                        acc.at[pl.ds(base, RPT)])

    @pl.when(sid == NS - 1)
    def _():
        pltpu.sync_copy(zeros_hbm.at[pl.ds(base, RPT_LAST)],
                        acc.at[pl.ds(base, RPT_LAST)])

    plsc.subcore_barrier()

    def step(g, carry):
        cs = [(3 * g + b) * NW + wid for b in range(3)]

        for (si, di, ro, se), c in zip(sets, cs):
            @pl.when(c < NCHUNK)
            def _(si=si, di=di, ro=ro, se=se, c=c):
                off = pl.multiple_of(c * CHUNK, CHUNK)
                pltpu.sync_copy(src_hbm.at[pl.ds(off, CHUNK)], si)
                pltpu.sync_copy(dst_hbm.at[pl.ds(off, CHUNK)], di)
                pltpu.async_copy(h_hbm.at[si], ro, se)

        for (si, di, ro, se), c in zip(sets, cs):
            @pl.when(c < NCHUNK)
            def _(si=si, di=di, ro=ro, se=se, c=c):
                pltpu.make_async_copy(h_hbm.at[si], ro, se).wait()
                pltpu.sync_copy(ro, acc.at[di], add=True)

        return carry

    lax.fori_loop(0, (TPT + 2) // 3, step, 0)
    plsc.subcore_barrier()

    @pl.when(sid < NS - 1)
    def _():
        pltpu.sync_copy(acc.at[pl.ds(base, RPT)],
                        out_hbm.at[pl.ds(cid * N + base, RPT)])

    @pl.when(sid == NS - 1)
    def _():
        pltpu.sync_copy(acc.at[pl.ds(base, RPT_LAST)],
                        out_hbm.at[pl.ds(cid * N + base, RPT_LAST)])


_seg_sum = pl.kernel(
    _seg_sum_body,
    out_type=jax.ShapeDtypeStruct((NC * N, D), jnp.float32),
    mesh=plsc.VectorSubcoreMesh(core_axis_name="c", subcore_axis_name="s"),
    scratch_types=[
        pltpu.VMEM((CHUNK,), jnp.int32),
        pltpu.VMEM((CHUNK,), jnp.int32),
        pltpu.VMEM((CHUNK, D), jnp.float32),
        pltpu.VMEM((CHUNK,), jnp.int32),
        pltpu.VMEM((CHUNK,), jnp.int32),
        pltpu.VMEM((CHUNK, D), jnp.float32),
        pltpu.VMEM((CHUNK,), jnp.int32),
        pltpu.VMEM((CHUNK,), jnp.int32),
        pltpu.VMEM((CHUNK, D), jnp.float32),
        pltpu.VMEM_SHARED((N, D), jnp.float32),
        pltpu.SemaphoreType.DMA,
        pltpu.SemaphoreType.DMA,
        pltpu.SemaphoreType.DMA,
    ],
)


def _bn(a, g, b):
    m = jnp.mean(a, axis=0)
    v = jnp.mean((a - m) * (a - m), axis=0)
    return (a - m) * lax.rsqrt(v + BN_EPS) * g + b


def _dense_body(h_ref, part_ref, w1_ref, b1_ref, gi_ref, bi_ref,
                w2_ref, b2_ref, go_ref, bo_ref, o_ref, *, relu_out):
    s = h_ref[...] + part_ref[:N] + part_ref[N:]
    a = jnp.dot(s, w1_ref[...], preferred_element_type=jnp.float32)
    a = a + b1_ref[...]
    a = jnp.maximum(_bn(a, gi_ref[...], bi_ref[...]), 0.0)
    o = jnp.dot(a, w2_ref[...], preferred_element_type=jnp.float32)
    o = o + b2_ref[...]
    o = _bn(o, go_ref[...], bo_ref[...])
    if relu_out:
        o = jnp.maximum(o, 0.0)
    o_ref[...] = o


def _dense(h, part, w1, b1, gi, bi, w2, b2, go, bo, relu_out):
    return pl.pallas_call(
        functools.partial(_dense_body, relu_out=relu_out),
        out_shape=jax.ShapeDtypeStruct((N, D), jnp.float32),
    )(h, part, w1, b1, gi, bi, w2, b2, go, bo)


def kernel(x, edge_index, batch,
           w1_0, b1_0, gi_0, bi_0, w2_0, b2_0, go_0, bo_0,
           w1_1, b1_1, gi_1, bi_1, w2_1, b2_1, go_1, bo_1,
           w1_2, b1_2, gi_2, bi_2, w2_2, b2_2, go_2, bo_2):
    src = edge_index[0]
    dst = edge_index[1]
    zeros = jnp.zeros((N, D), jnp.float32)

    params = [
        (w1_0, b1_0, gi_0, bi_0, w2_0, b2_0, go_0, bo_0),
        (w1_1, b1_1, gi_1, bi_1, w2_1, b2_1, go_1, bo_1),
        (w1_2, b1_2, gi_2, bi_2, w2_2, b2_2, go_2, bo_2),
    ]

    h = x
    for l in range(3):
        part = _seg_sum(h, src, dst, zeros)
        h = _dense(h, part, *params[l], relu_out=(l < 2))
    return h


# speedup vs baseline: 3.2573x; 1.0283x over previous
import functools

import jax
import jax.numpy as jnp
from jax import lax
from jax.experimental import pallas as pl
from jax.experimental.pallas import tpu as pltpu
from jax.experimental.pallas import tpu_sc as plsc

N = 10000
E = 320000
D = 128
BN_EPS = 1e-5

NC = 2
NS = 16
NW = NC * NS

CHUNK = 128
NCHUNK = E // CHUNK
TPT = (NCHUNK + NW - 1) // NW
RPT = 624
RPT_LAST = N - 15 * RPT


def _seg_sum_body(h_hbm, src_hbm, dst_hbm, zeros_hbm, out_hbm,
                  sidx0, didx0, rows0, sidx1, didx1, rows1,
                  sidx2, didx2, rows2, acc, sem0, sem1, sem2):
    cid = lax.axis_index("c")
    sid = lax.axis_index("s")
    wid = cid * NS + sid
    sets = [(sidx0, didx0, rows0, sem0),
            (sidx1, didx1, rows1, sem1),
            (sidx2, didx2, rows2, sem2)]

    base = sid * RPT

    @pl.when(sid < NS - 1)
    def _():
        pltpu.sync_copy(zeros_hbm.at[pl.ds(base, RPT)],
                        acc.at[pl.ds(base, RPT)])

    @pl.when(sid == NS - 1)
    def _():
        pltpu.sync_copy(zeros_hbm.at[pl.ds(base, RPT_LAST)],
                        acc.at[pl.ds(base, RPT_LAST)])

    plsc.subcore_barrier()

    def issue(si, di, ro, se, c):
        @pl.when(c < NCHUNK)
        def _():
            off = pl.multiple_of(c * CHUNK, CHUNK)
            pltpu.sync_copy(src_hbm.at[pl.ds(off, CHUNK)], si)
            pltpu.sync_copy(dst_hbm.at[pl.ds(off, CHUNK)], di)
            pltpu.async_copy(h_hbm.at[si], ro, se)

    for b, (si, di, ro, se) in enumerate(sets):
        issue(si, di, ro, se, b * NW + wid)

    def step(g, carry):
        for b, (si, di, ro, se) in enumerate(sets):
            c = (3 * g + b) * NW + wid

            @pl.when(c < NCHUNK)
            def _(si=si, di=di, ro=ro, se=se, c=c):
                pltpu.make_async_copy(h_hbm.at[si], ro, se).wait()
                pltpu.sync_copy(ro, acc.at[di], add=True)

            issue(si, di, ro, se, (3 * (g + 1) + b) * NW + wid)

        return carry

    lax.fori_loop(0, (TPT + 2) // 3, step, 0)
    plsc.subcore_barrier()

    @pl.when(sid < NS - 1)
    def _():
        pltpu.sync_copy(acc.at[pl.ds(base, RPT)],
                        out_hbm.at[pl.ds(cid * N + base, RPT)])

    @pl.when(sid == NS - 1)
    def _():
        pltpu.sync_copy(acc.at[pl.ds(base, RPT_LAST)],
                        out_hbm.at[pl.ds(cid * N + base, RPT_LAST)])


_seg_sum = pl.kernel(
    _seg_sum_body,
    out_type=jax.ShapeDtypeStruct((NC * N, D), jnp.float32),
    mesh=plsc.VectorSubcoreMesh(core_axis_name="c", subcore_axis_name="s"),
    scratch_types=[
        pltpu.VMEM((CHUNK,), jnp.int32),
        pltpu.VMEM((CHUNK,), jnp.int32),
        pltpu.VMEM((CHUNK, D), jnp.float32),
        pltpu.VMEM((CHUNK,), jnp.int32),
        pltpu.VMEM((CHUNK,), jnp.int32),
        pltpu.VMEM((CHUNK, D), jnp.float32),
        pltpu.VMEM((CHUNK,), jnp.int32),
        pltpu.VMEM((CHUNK,), jnp.int32),
        pltpu.VMEM((CHUNK, D), jnp.float32),
        pltpu.VMEM_SHARED((N, D), jnp.float32),
        pltpu.SemaphoreType.DMA,
        pltpu.SemaphoreType.DMA,
        pltpu.SemaphoreType.DMA,
    ],
)


def _bn(a, g, b):
    m = jnp.mean(a, axis=0)
    v = jnp.mean((a - m) * (a - m), axis=0)
    return (a - m) * lax.rsqrt(v + BN_EPS) * g + b


def _dense_body(h_ref, part_ref, w1_ref, b1_ref, gi_ref, bi_ref,
                w2_ref, b2_ref, go_ref, bo_ref, o_ref, *, relu_out):
    s = h_ref[...] + part_ref[:N] + part_ref[N:]
    a = jnp.dot(s, w1_ref[...], preferred_element_type=jnp.float32)
    a = a + b1_ref[...]
    a = jnp.maximum(_bn(a, gi_ref[...], bi_ref[...]), 0.0)
    o = jnp.dot(a, w2_ref[...], preferred_element_type=jnp.float32)
    o = o + b2_ref[...]
    o = _bn(o, go_ref[...], bo_ref[...])
    if relu_out:
        o = jnp.maximum(o, 0.0)
    o_ref[...] = o


def _dense(h, part, w1, b1, gi, bi, w2, b2, go, bo, relu_out):
    return pl.pallas_call(
        functools.partial(_dense_body, relu_out=relu_out),
        out_shape=jax.ShapeDtypeStruct((N, D), jnp.float32),
    )(h, part, w1, b1, gi, bi, w2, b2, go, bo)


def kernel(x, edge_index, batch,
           w1_0, b1_0, gi_0, bi_0, w2_0, b2_0, go_0, bo_0,
           w1_1, b1_1, gi_1, bi_1, w2_1, b2_1, go_1, bo_1,
           w1_2, b1_2, gi_2, bi_2, w2_2, b2_2, go_2, bo_2):
    src = edge_index[0]
    dst = edge_index[1]
    zeros = jnp.zeros((N, D), jnp.float32)

    params = [
        (w1_0, b1_0, gi_0, bi_0, w2_0, b2_0, go_0, bo_0),
        (w1_1, b1_1, gi_1, bi_1, w2_1, b2_1, go_1, bo_1),
        (w1_2, b1_2, gi_2, bi_2, w2_2, b2_2, go_2, bo_2),
    ]

    h = x
    for l in range(3):
        part = _seg_sum(h, src, dst, zeros)
        h = _dense(h, part, *params[l], relu_out=(l < 2))
    return h


# speedup vs baseline: 4.6116x; 1.4158x over previous
import functools

import jax
import jax.numpy as jnp
from jax import lax
from jax.experimental import pallas as pl
from jax.experimental.pallas import tpu as pltpu
from jax.experimental.pallas import tpu_sc as plsc

N = 10000
E = 320000
D = 128
BN_EPS = 1e-5

NC = 2
NS = 16
NW = NC * NS

CHUNK = 128
NCHUNK = E // CHUNK
TPT = (NCHUNK + NW - 1) // NW
RPT = 624
RPT_LAST = N - 15 * RPT


def _seg_sum_body(h_hbm, src_hbm, dst_hbm, zeros_hbm, out_hbm,
                  sA0, dA0, sB0, dB0, rows0,
                  sA1, dA1, sB1, dB1, rows1,
                  sA2, dA2, sB2, dB2, rows2,
                  acc, sg0, sg1, sg2, sx0, sx1, sx2):
    cid = lax.axis_index("c")
    sid = lax.axis_index("s")
    wid = cid * NS + sid
    rows = [rows0, rows1, rows2]
    sg = [sg0, sg1, sg2]
    sx = [sx0, sx1, sx2]
    pairA = [(sA0, dA0), (sA1, dA1), (sA2, dA2)]
    pairB = [(sB0, dB0), (sB1, dB1), (sB2, dB2)]

    base = sid * RPT

    @pl.when(sid < NS - 1)
    def _():
        pltpu.sync_copy(zeros_hbm.at[pl.ds(base, RPT)],
                        acc.at[pl.ds(base, RPT)])

    @pl.when(sid == NS - 1)
    def _():
        pltpu.sync_copy(zeros_hbm.at[pl.ds(base, RPT_LAST)],
                        acc.at[pl.ds(base, RPT_LAST)])

    plsc.subcore_barrier()

    for b in range(3):
        si, di = pairA[b]
        c = b * NW + wid
        off = pl.multiple_of(c * CHUNK, CHUNK)
        pltpu.sync_copy(src_hbm.at[pl.ds(off, CHUNK)], si)
        pltpu.sync_copy(dst_hbm.at[pl.ds(off, CHUNK)], di)
        pltpu.async_copy(h_hbm.at[si], rows[b], sg[b])
    for b in range(3):
        si, di = pairB[b]
        c = (3 + b) * NW + wid
        off = pl.multiple_of(c * CHUNK, CHUNK)
        pltpu.async_copy(src_hbm.at[pl.ds(off, CHUNK)], si, sx[b])
        pltpu.async_copy(dst_hbm.at[pl.ds(off, CHUNK)], di, sx[b])

    def half(g, cur, nxt):
        for b in range(3):
            si, di = cur[b]
            sn, dn = nxt[b]
            c = (3 * g + b) * NW + wid
            c1 = (3 * (g + 1) + b) * NW + wid
            c2 = (3 * (g + 2) + b) * NW + wid

            @pl.when(c < NCHUNK)
            def _(si=si, di=di, b=b, c=c):
                pltpu.make_async_copy(h_hbm.at[si], rows[b], sg[b]).wait()
                pltpu.sync_copy(rows[b], acc.at[di], add=True)

            @pl.when(c2 < NCHUNK)
            def _(si=si, di=di, b=b, c2=c2):
                off = pl.multiple_of(c2 * CHUNK, CHUNK)
                pltpu.async_copy(src_hbm.at[pl.ds(off, CHUNK)], si, sx[b])
                pltpu.async_copy(dst_hbm.at[pl.ds(off, CHUNK)], di, sx[b])

            @pl.when(c1 < NCHUNK)
            def _(sn=sn, dn=dn, b=b):
                pltpu.make_async_copy(src_hbm.at[pl.ds(0, CHUNK)], sn,
                                      sx[b]).wait()
                pltpu.make_async_copy(src_hbm.at[pl.ds(0, CHUNK)], dn,
                                      sx[b]).wait()
                pltpu.async_copy(h_hbm.at[sn], rows[b], sg[b])

    def step(gg, carry):
        half(2 * gg, pairA, pairB)
        half(2 * gg + 1, pairB, pairA)
        return carry

    lax.fori_loop(0, 14, step, 0)
    plsc.subcore_barrier()

    @pl.when(sid < NS - 1)
    def _():
        pltpu.sync_copy(acc.at[pl.ds(base, RPT)],
                        out_hbm.at[pl.ds(cid * N + base, RPT)])

    @pl.when(sid == NS - 1)
    def _():
        pltpu.sync_copy(acc.at[pl.ds(base, RPT_LAST)],
                        out_hbm.at[pl.ds(cid * N + base, RPT_LAST)])


_seg_sum = pl.kernel(
    _seg_sum_body,
    out_type=jax.ShapeDtypeStruct((NC * N, D), jnp.float32),
    mesh=plsc.VectorSubcoreMesh(core_axis_name="c", subcore_axis_name="s"),
    scratch_types=(
        ([pltpu.VMEM((CHUNK,), jnp.int32)] * 4
         + [pltpu.VMEM((CHUNK, D), jnp.float32)]) * 3
        + [pltpu.VMEM_SHARED((N, D), jnp.float32)]
        + [pltpu.SemaphoreType.DMA] * 6
    ),
)


def _bn(a, g, b):
    m = jnp.mean(a, axis=0)
    v = jnp.mean((a - m) * (a - m), axis=0)
    return (a - m) * lax.rsqrt(v + BN_EPS) * g + b


def _dense_body(h_ref, part_ref, w1_ref, b1_ref, gi_ref, bi_ref,
                w2_ref, b2_ref, go_ref, bo_ref, o_ref, *, relu_out):
    s = h_ref[...] + part_ref[:N] + part_ref[N:]
    a = jnp.dot(s, w1_ref[...], preferred_element_type=jnp.float32)
    a = a + b1_ref[...]
    a = jnp.maximum(_bn(a, gi_ref[...], bi_ref[...]), 0.0)
    o = jnp.dot(a, w2_ref[...], preferred_element_type=jnp.float32)
    o = o + b2_ref[...]
    o = _bn(o, go_ref[...], bo_ref[...])
    if relu_out:
        o = jnp.maximum(o, 0.0)
    o_ref[...] = o


def _dense(h, part, w1, b1, gi, bi, w2, b2, go, bo, relu_out):
    return pl.pallas_call(
        functools.partial(_dense_body, relu_out=relu_out),
        out_shape=jax.ShapeDtypeStruct((N, D), jnp.float32),
    )(h, part, w1, b1, gi, bi, w2, b2, go, bo)


def kernel(x, edge_index, batch,
           w1_0, b1_0, gi_0, bi_0, w2_0, b2_0, go_0, bo_0,
           w1_1, b1_1, gi_1, bi_1, w2_1, b2_1, go_1, bo_1,
           w1_2, b1_2, gi_2, bi_2, w2_2, b2_2, go_2, bo_2):
    src = edge_index[0]
    dst = edge_index[1]
    zeros = jnp.zeros((N, D), jnp.float32)

    params = [
        (w1_0, b1_0, gi_0, bi_0, w2_0, b2_0, go_0, bo_0),
        (w1_1, b1_1, gi_1, bi_1, w2_1, b2_1, go_1, bo_1),
        (w1_2, b1_2, gi_2, bi_2, w2_2, b2_2, go_2, bo_2),
    ]

    h = x
    for l in range(3):
        part = _seg_sum(h, src, dst, zeros)
        h = _dense(h, part, *params[l], relu_out=(l < 2))
    return h


# speedup vs baseline: 4.6643x; 1.0114x over previous
import functools

import jax
import jax.numpy as jnp
from jax import lax
from jax.experimental import pallas as pl
from jax.experimental.pallas import tpu as pltpu
from jax.experimental.pallas import tpu_sc as plsc

N = 10000
E = 320000
D = 128
BN_EPS = 1e-5

NC = 2
NS = 16
NW = NC * NS

CHUNK = 128
NCHUNK = E // CHUNK
TPT = (NCHUNK + NW - 1) // NW
RPT = 624
RPT_LAST = N - 15 * RPT


def _seg_sum_body(h_hbm, src_hbm, dst_hbm, zeros_hbm, out_hbm,
                  sA0, dA0, sB0, dB0, rows0,
                  sA1, dA1, sB1, dB1, rows1,
                  sA2, dA2, sB2, dB2, rows2,
                  acc, sg0, sg1, sg2, sx0, sx1, sx2):
    cid = lax.axis_index("c")
    sid = lax.axis_index("s")
    wid = cid * NS + sid
    rows = [rows0, rows1, rows2]
    sg = [sg0, sg1, sg2]
    sx = [sx0, sx1, sx2]
    pairA = [(sA0, dA0), (sA1, dA1), (sA2, dA2)]
    pairB = [(sB0, dB0), (sB1, dB1), (sB2, dB2)]

    for pair, g0 in ((pairA, 0), (pairB, 1)):
        for b in range(3):
            si, di = pair[b]
            c = (3 * g0 + b) * NW + wid
            off = pl.multiple_of(c * CHUNK, CHUNK)
            pltpu.async_copy(src_hbm.at[pl.ds(off, CHUNK)], si, sx[b])
            pltpu.async_copy(dst_hbm.at[pl.ds(off, CHUNK)], di, sx[b])

    base = sid * RPT

    @pl.when(sid < NS - 1)
    def _():
        pltpu.sync_copy(zeros_hbm.at[pl.ds(base, RPT)],
                        acc.at[pl.ds(base, RPT)])

    @pl.when(sid == NS - 1)
    def _():
        pltpu.sync_copy(zeros_hbm.at[pl.ds(base, RPT_LAST)],
                        acc.at[pl.ds(base, RPT_LAST)])

    for b in range(3):
        si, di = pairA[b]
        pltpu.make_async_copy(src_hbm.at[pl.ds(0, CHUNK)], si, sx[b]).wait()
        pltpu.make_async_copy(src_hbm.at[pl.ds(0, CHUNK)], di, sx[b]).wait()
        pltpu.async_copy(h_hbm.at[si], rows[b], sg[b])

    plsc.subcore_barrier()

    def half(g, cur, nxt):
        for b in range(3):
            si, di = cur[b]
            sn, dn = nxt[b]
            c = (3 * g + b) * NW + wid
            c1 = (3 * (g + 1) + b) * NW + wid
            c2 = (3 * (g + 2) + b) * NW + wid

            @pl.when(c < NCHUNK)
            def _(si=si, di=di, b=b, c=c):
                pltpu.make_async_copy(h_hbm.at[si], rows[b], sg[b]).wait()
                pltpu.sync_copy(rows[b], acc.at[di], add=True)

            @pl.when(c2 < NCHUNK)
            def _(si=si, di=di, b=b, c2=c2):
                off = pl.multiple_of(c2 * CHUNK, CHUNK)
                pltpu.async_copy(src_hbm.at[pl.ds(off, CHUNK)], si, sx[b])
                pltpu.async_copy(dst_hbm.at[pl.ds(off, CHUNK)], di, sx[b])

            @pl.when(c1 < NCHUNK)
            def _(sn=sn, dn=dn, b=b):
                pltpu.make_async_copy(src_hbm.at[pl.ds(0, CHUNK)], sn,
                                      sx[b]).wait()
                pltpu.make_async_copy(src_hbm.at[pl.ds(0, CHUNK)], dn,
                                      sx[b]).wait()
                pltpu.async_copy(h_hbm.at[sn], rows[b], sg[b])

    def step(gg, carry):
        half(2 * gg, pairA, pairB)
        half(2 * gg + 1, pairB, pairA)
        return carry

    lax.fori_loop(0, 14, step, 0)
    plsc.subcore_barrier()

    @pl.when(sid < NS - 1)
    def _():
        pltpu.sync_copy(acc.at[pl.ds(base, RPT)],
                        out_hbm.at[pl.ds(cid * N + base, RPT)])

    @pl.when(sid == NS - 1)
    def _():
        pltpu.sync_copy(acc.at[pl.ds(base, RPT_LAST)],
                        out_hbm.at[pl.ds(cid * N + base, RPT_LAST)])


_seg_sum = pl.kernel(
    _seg_sum_body,
    out_type=jax.ShapeDtypeStruct((NC * N, D), jnp.float32),
    mesh=plsc.VectorSubcoreMesh(core_axis_name="c", subcore_axis_name="s"),
    scratch_types=(
        ([pltpu.VMEM((CHUNK,), jnp.int32)] * 4
         + [pltpu.VMEM((CHUNK, D), jnp.float32)]) * 3
        + [pltpu.VMEM_SHARED((N, D), jnp.float32)]
        + [pltpu.SemaphoreType.DMA] * 6
    ),
)


def _bn(a, g, b):
    m = jnp.mean(a, axis=0)
    v = jnp.mean((a - m) * (a - m), axis=0)
    return (a - m) * lax.rsqrt(v + BN_EPS) * g + b


def _dense_body(h_ref, part_ref, w1_ref, b1_ref, gi_ref, bi_ref,
                w2_ref, b2_ref, go_ref, bo_ref, o_ref, *, relu_out):
    s = h_ref[...] + part_ref[:N] + part_ref[N:]
    a = jnp.dot(s, w1_ref[...], preferred_element_type=jnp.float32)
    a = a + b1_ref[...]
    a = jnp.maximum(_bn(a, gi_ref[...], bi_ref[...]), 0.0)
    o = jnp.dot(a, w2_ref[...], preferred_element_type=jnp.float32)
    o = o + b2_ref[...]
    o = _bn(o, go_ref[...], bo_ref[...])
    if relu_out:
        o = jnp.maximum(o, 0.0)
    o_ref[...] = o


def _dense(h, part, w1, b1, gi, bi, w2, b2, go, bo, relu_out):
    return pl.pallas_call(
        functools.partial(_dense_body, relu_out=relu_out),
        out_shape=jax.ShapeDtypeStruct((N, D), jnp.float32),
    )(h, part, w1, b1, gi, bi, w2, b2, go, bo)


def kernel(x, edge_index, batch,
           w1_0, b1_0, gi_0, bi_0, w2_0, b2_0, go_0, bo_0,
           w1_1, b1_1, gi_1, bi_1, w2_1, b2_1, go_1, bo_1,
           w1_2, b1_2, gi_2, bi_2, w2_2, b2_2, go_2, bo_2):
    src = edge_index[0]
    dst = edge_index[1]
    zeros = jnp.zeros((N, D), jnp.float32)

    params = [
        (w1_0, b1_0, gi_0, bi_0, w2_0, b2_0, go_0, bo_0),
        (w1_1, b1_1, gi_1, bi_1, w2_1, b2_1, go_1, bo_1),
        (w1_2, b1_2, gi_2, bi_2, w2_2, b2_2, go_2, bo_2),
    ]

    h = x
    for l in range(3):
        part = _seg_sum(h, src, dst, zeros)
        h = _dense(h, part, *params[l], relu_out=(l < 2))
    return h


# speedup vs baseline: 4.7605x; 1.0206x over previous
import functools

import jax
import jax.numpy as jnp
from jax import lax
from jax.experimental import pallas as pl
from jax.experimental.pallas import tpu as pltpu
from jax.experimental.pallas import tpu_sc as plsc

N = 10000
E = 320000
D = 128
BN_EPS = 1e-5

NC = 2
NS = 16
NW = NC * NS

CHUNK = 128
NCHUNK = E // CHUNK
TPT = (NCHUNK + NW - 1) // NW
RPT = 624
RPT_LAST = N - 15 * RPT


def _seg_sum_body(h_hbm, src_hbm, dst_hbm, zeros_hbm, out_hbm,
                  sA0, dA0, sB0, dB0, rows0,
                  sA1, dA1, sB1, dB1, rows1,
                  sA2, dA2, sB2, dB2, rows2,
                  acc, sg0, sg1, sg2, sx0, sx1, sx2):
    cid = lax.axis_index("c")
    sid = lax.axis_index("s")
    wid = cid * NS + sid
    rows = [rows0, rows1, rows2]
    sg = [sg0, sg1, sg2]
    sx = [sx0, sx1, sx2]
    pairA = [(sA0, dA0), (sA1, dA1), (sA2, dA2)]
    pairB = [(sB0, dB0), (sB1, dB1), (sB2, dB2)]

    for pair, g0 in ((pairA, 0), (pairB, 1)):
        for b in range(3):
            si, di = pair[b]
            c = (3 * g0 + b) * NW + wid
            off = pl.multiple_of(c * CHUNK, CHUNK)
            pltpu.async_copy(src_hbm.at[pl.ds(off, CHUNK)], si, sx[b])
            pltpu.async_copy(dst_hbm.at[pl.ds(off, CHUNK)], di, sx[b])

    base = sid * RPT

    @pl.when(sid < NS - 1)
    def _():
        pltpu.sync_copy(zeros_hbm.at[pl.ds(base, RPT)],
                        acc.at[pl.ds(base, RPT)])

    @pl.when(sid == NS - 1)
    def _():
        pltpu.sync_copy(zeros_hbm.at[pl.ds(base, RPT_LAST)],
                        acc.at[pl.ds(base, RPT_LAST)])

    for b in range(3):
        si, di = pairA[b]
        pltpu.make_async_copy(src_hbm.at[pl.ds(0, CHUNK)], si, sx[b]).wait()
        pltpu.make_async_copy(src_hbm.at[pl.ds(0, CHUNK)], di, sx[b]).wait()
        pltpu.async_copy(h_hbm.at[si], rows[b], sg[b])

    plsc.subcore_barrier()

    def half(g, cur, nxt):
        for b in range(3):
            si, di = cur[b]
            sn, dn = nxt[b]
            c = (3 * g + b) * NW + wid
            c1 = (3 * (g + 1) + b) * NW + wid
            c2 = (3 * (g + 2) + b) * NW + wid

            @pl.when(c < NCHUNK)
            def _(si=si, di=di, b=b, c=c):
                pltpu.make_async_copy(h_hbm.at[si], rows[b], sg[b]).wait()
                pltpu.sync_copy(rows[b], acc.at[di], add=True)

            @pl.when(c2 < NCHUNK)
            def _(si=si, di=di, b=b, c2=c2):
                off = pl.multiple_of(c2 * CHUNK, CHUNK)
                pltpu.async_copy(src_hbm.at[pl.ds(off, CHUNK)], si, sx[b])
                pltpu.async_copy(dst_hbm.at[pl.ds(off, CHUNK)], di, sx[b])

            @pl.when(c1 < NCHUNK)
            def _(sn=sn, dn=dn, b=b):
                pltpu.make_async_copy(src_hbm.at[pl.ds(0, CHUNK)], sn,
                                      sx[b]).wait()
                pltpu.make_async_copy(src_hbm.at[pl.ds(0, CHUNK)], dn,
                                      sx[b]).wait()
                pltpu.async_copy(h_hbm.at[sn], rows[b], sg[b])

    def step(gg, carry):
        half(2 * gg, pairA, pairB)
        half(2 * gg + 1, pairB, pairA)
        return carry

    lax.fori_loop(0, 14, step, 0)
    plsc.subcore_barrier()

    @pl.when(sid < NS - 1)
    def _():
        pltpu.sync_copy(acc.at[pl.ds(base, RPT)],
                        out_hbm.at[pl.ds(cid * N + base, RPT)])

    @pl.when(sid == NS - 1)
    def _():
        pltpu.sync_copy(acc.at[pl.ds(base, RPT_LAST)],
                        out_hbm.at[pl.ds(cid * N + base, RPT_LAST)])


_seg_sum = pl.kernel(
    _seg_sum_body,
    out_type=jax.ShapeDtypeStruct((NC * N, D), jnp.float32),
    mesh=plsc.VectorSubcoreMesh(core_axis_name="c", subcore_axis_name="s"),
    scratch_types=(
        ([pltpu.VMEM((CHUNK,), jnp.int32)] * 4
         + [pltpu.VMEM((CHUNK, D), jnp.float32)]) * 3
        + [pltpu.VMEM_SHARED((N, D), jnp.float32)]
        + [pltpu.SemaphoreType.DMA] * 6
    ),
)


def _bn(a, g, b):
    m = jnp.mean(a, axis=0)
    v = jnp.mean(a * a, axis=0) - m * m
    return (a - m) * lax.rsqrt(v + BN_EPS) * g + b


def _dense_body(h_ref, part_ref, w1_ref, b1_ref, gi_ref, bi_ref,
                w2_ref, b2_ref, go_ref, bo_ref, o_ref, *, relu_out):
    s = h_ref[...] + part_ref[:N] + part_ref[N:]
    a = jnp.dot(s, w1_ref[...], preferred_element_type=jnp.float32)
    a = a + b1_ref[...]
    a = jnp.maximum(_bn(a, gi_ref[...], bi_ref[...]), 0.0)
    o = jnp.dot(a, w2_ref[...], preferred_element_type=jnp.float32)
    o = o + b2_ref[...]
    o = _bn(o, go_ref[...], bo_ref[...])
    if relu_out:
        o = jnp.maximum(o, 0.0)
    o_ref[...] = o


def _dense(h, part, w1, b1, gi, bi, w2, b2, go, bo, relu_out):
    return pl.pallas_call(
        functools.partial(_dense_body, relu_out=relu_out),
        out_shape=jax.ShapeDtypeStruct((N, D), jnp.float32),
    )(h, part, w1, b1, gi, bi, w2, b2, go, bo)


def kernel(x, edge_index, batch,
           w1_0, b1_0, gi_0, bi_0, w2_0, b2_0, go_0, bo_0,
           w1_1, b1_1, gi_1, bi_1, w2_1, b2_1, go_1, bo_1,
           w1_2, b1_2, gi_2, bi_2, w2_2, b2_2, go_2, bo_2):
    src = edge_index[0]
    dst = edge_index[1]
    zeros = jnp.zeros((N, D), jnp.float32)

    params = [
        (w1_0, b1_0, gi_0, bi_0, w2_0, b2_0, go_0, bo_0),
        (w1_1, b1_1, gi_1, bi_1, w2_1, b2_1, go_1, bo_1),
        (w1_2, b1_2, gi_2, bi_2, w2_2, b2_2, go_2, bo_2),
    ]

    h = x
    for l in range(3):
        part = _seg_sum(h, src, dst, zeros)
        h = _dense(h, part, *params[l], relu_out=(l < 2))
    return h
